# Initial kernel scaffold; baseline (speedup 1.0000x reference)
import jax
import jax.numpy as jnp
from jax.experimental import pallas as pl


def kernel(feat, clusters, nbr_idx, proj_W, proj_g, proj_b, lw_W, lw_g, lw_b, wgt_W, adp_W, fuse_W, fuse_g, fuse_b, conv1_W, bn1_g, bn1_b, conv2_W, bn2_g, bn2_b):
    raise NotImplementedError("write your pallas kernel here")


# speedup vs baseline: 1.0010x; 1.0010x over previous
import jax
import jax.numpy as jnp
from jax.experimental import pallas as pl
from jax.experimental.pallas import tpu as pltpu

N = 50000
C = 128
K = 4096
NBR = 27


def _bn(x, g, b):
    m = jnp.mean(x, axis=0)
    v = jnp.var(x, axis=0)
    return (x - m) / jnp.sqrt(v + 1e-5) * g + b


def _seg_sum(x, ids):
    return jax.ops.segment_sum(x, ids, num_segments=K)


def _seg_mean(x, ids):
    s = _seg_sum(x, ids)
    cnt = jax.ops.segment_sum(jnp.ones((x.shape[0],), x.dtype), ids, num_segments=K)
    return s / jnp.clip(cnt, 1.0)[:, None]


def _subm_conv(x, W, nbr):
    out = jnp.zeros((x.shape[0], W.shape[2]), x.dtype)
    for k in range(W.shape[0]):
        out = out + x[nbr[:, k]] @ W[k]
    return out


def _final_relu_add_kernel(h_ref, res_ref, out_ref):
    out_ref[...] = jnp.maximum(h_ref[...] + res_ref[...], 0.0)


def _final_relu_add(h, res):
    n = h.shape[0]
    blk = 2000
    return pl.pallas_call(
        _final_relu_add_kernel,
        grid=(n // blk,),
        in_specs=[
            pl.BlockSpec((blk, C), lambda i: (i, 0)),
            pl.BlockSpec((blk, C), lambda i: (i, 0)),
        ],
        out_specs=pl.BlockSpec((blk, C), lambda i: (i, 0)),
        out_shape=jax.ShapeDtypeStruct((n, C), h.dtype),
    )(h, res)


def kernel(feat, clusters, nbr_idx, proj_W, proj_g, proj_b, lw_W, lw_g, lw_b, wgt_W, adp_W, fuse_W, fuse_g, fuse_b, conv1_W, bn1_g, bn1_b, conv2_W, bn2_g, bn2_b):
    feats = []
    for i in range(3):
        cl = clusters[i]
        pw = jax.nn.relu(_bn(feat @ lw_W[i], lw_g[i], lw_b[i]))
        pw = pw - _seg_mean(pw, cl)[cl]
        pw = pw @ wgt_W[i]
        pw = jnp.exp(pw - jnp.max(pw))
        pw = pw / (_seg_sum(pw, cl)[cl] + 1e-6)
        pf = jax.nn.relu(_bn(feat @ proj_W[i], proj_g[i], proj_b[i])) * pw
        pf = _seg_sum(pf, cl)[cl]
        feats.append(pf)
    adp = jax.nn.softmax(feat @ adp_W, axis=1)
    fs = jnp.stack(feats, axis=1)
    agg = jnp.einsum('ln,lnc->lc', adp, fs)
    f = jax.nn.relu(_bn(feat @ proj_W[3], proj_g[3], proj_b[3]))
    f = jnp.concatenate([f, agg], axis=1)
    f = jax.nn.relu(_bn(f @ fuse_W, fuse_g, fuse_b)) + feat
    res = f
    h = _subm_conv(f, conv1_W, nbr_idx)
    h = jax.nn.relu(_bn(h, bn1_g, bn1_b))
    h = _subm_conv(h, conv2_W, nbr_idx)
    h = _bn(h, bn2_g, bn2_b)
    return _final_relu_add(h, res)


# speedup vs baseline: 1.9695x; 1.9676x over previous
import functools

import jax
import jax.numpy as jnp
from jax import lax
from jax.experimental import pallas as pl
from jax.experimental.pallas import tpu as pltpu
from jax.experimental.pallas import tpu_sc as plsc

N = 50000
C = 128
K = 4096
NBR = 27
NPAD = 50176
NGRP = NPAD // 128
NW = 32
GPW = (NGRP + NW - 1) // NW
KR = K // 16

_MESH = plsc.VectorSubcoreMesh(core_axis_name="c", subcore_axis_name="s")
_f32 = jnp.float32


def _wid():
    return lax.axis_index("s") * 2 + lax.axis_index("c")


@functools.partial(
    pl.kernel,
    out_type=jax.ShapeDtypeStruct((2, 3, K, C), _f32),
    mesh=_MESH,
    scratch_types=(
        pltpu.VMEM((128,), jnp.int32),
        pltpu.VMEM((128, C), _f32),
        pltpu.VMEM_SHARED((K, C), _f32),
        pltpu.VMEM_SHARED((K, C), _f32),
        pltpu.VMEM_SHARED((K, C), _f32),
    ),
)
def _seg_scatter3(xs, ids, ztab, parts, idxb, rowb, t0, t1, t2):
    tabs = (t0, t1, t2)
    c = lax.axis_index("c")
    s = lax.axis_index("s")
    wid = _wid()
    for i in range(3):
        pltpu.sync_copy(ztab.at[pl.ds(s * KR, KR)], tabs[i].at[pl.ds(s * KR, KR)])
    plsc.subcore_barrier()

    def g_body(g, _):
        gid = wid + NW * g

        @pl.when(gid < NGRP)
        def _():
            base = gid * 128
            for i in range(3):
                pltpu.sync_copy(ids.at[i, gid], idxb)
                pltpu.sync_copy(xs.at[i, pl.ds(base, 128)], rowb)
                pltpu.sync_copy(rowb, tabs[i].at[idxb], add=True)
        return 0

    lax.fori_loop(0, GPW, g_body, 0)
    plsc.subcore_barrier()
    for i in range(3):
        pltpu.sync_copy(tabs[i].at[pl.ds(s * KR, KR)], parts.at[c, i, pl.ds(s * KR, KR)])


def _seg_counts3(ids, ztab, vals_ones3):
    parts = _seg_scatter3(vals_ones3, ids, ztab)
    return (parts[0] + parts[1])[..., 0]


@functools.partial(
    pl.kernel,
    out_type=jax.ShapeDtypeStruct((3, NPAD, C), _f32),
    mesh=_MESH,
    scratch_types=(
        pltpu.VMEM((128,), jnp.int32),
        pltpu.VMEM((128, C), _f32),
        pltpu.SemaphoreType.DMA,
    ),
)
def _seg_gather3(tabs, ids_off, out, idxb, rowb, sem):
    wid = _wid()

    def g_body(g, _):
        gid = wid + NW * g

        @pl.when(gid < NGRP)
        def _():
            base = gid * 128
            for i in range(3):
                pltpu.sync_copy(ids_off.at[i, gid], idxb)
                pltpu.async_copy(tabs.at[idxb], rowb, sem).wait()
                pltpu.sync_copy(rowb, out.at[i, pl.ds(base, 128)])
        return 0

    lax.fori_loop(0, GPW, g_body, 0)


@functools.partial(
    pl.kernel,
    out_type=jax.ShapeDtypeStruct((NPAD, C), _f32),
    mesh=_MESH,
    scratch_types=(
        pltpu.VMEM((128,), jnp.int32),
        pltpu.VMEM((128, C), _f32),
        pltpu.VMEM((128, C), _f32),
        pltpu.SemaphoreType.DMA,
    ),
)
def _conv_gather_add(yflat, nbr_off, out, idxb, accb, rowb, sem):
    wid = _wid()

    def g_body(g, _):
        gid = wid + NW * g

        @pl.when(gid < NGRP)
        def _():
            base = gid * 128
            pltpu.sync_copy(nbr_off.at[0, gid], idxb)
            pltpu.async_copy(yflat.at[idxb], accb, sem).wait()

            def k_body(k, _):
                pltpu.sync_copy(nbr_off.at[k, gid], idxb)
                pltpu.async_copy(yflat.at[idxb], rowb, sem).wait()

                def _acc_row(r, _):
                    for cc in range(8):
                        sl = pl.ds(cc * 16, 16)
                        accb[r, sl] = accb[r, sl] + rowb[r, sl]
                    return 0

                lax.fori_loop(0, 128, _acc_row, 0)
                return 0

            lax.fori_loop(1, NBR, k_body, 0)
            pltpu.sync_copy(accb, out.at[pl.ds(base, 128)])
        return 0

    lax.fori_loop(0, GPW, g_body, 0)


def _final_relu_add_kernel(h_ref, res_ref, out_ref):
    out_ref[...] = jnp.maximum(h_ref[...] + res_ref[...], 0.0)


def _final_relu_add(h, res):
    blk = 2000
    return pl.pallas_call(
        _final_relu_add_kernel,
        grid=(N // blk,),
        in_specs=[
            pl.BlockSpec((blk, C), lambda i: (i, 0)),
            pl.BlockSpec((blk, C), lambda i: (i, 0)),
        ],
        out_specs=pl.BlockSpec((blk, C), lambda i: (i, 0)),
        out_shape=jax.ShapeDtypeStruct((N, C), h.dtype),
    )(h, res)


def _bn(x, g, b, axis=0):
    m = jnp.mean(x, axis=axis, keepdims=True)
    v = jnp.var(x, axis=axis, keepdims=True)
    return (x - m) / jnp.sqrt(v + 1e-5) * g + b


def _pad_rows(x, npad):
    pads = [(0, 0)] * x.ndim
    pads[-2] = (0, npad - x.shape[-2])
    return jnp.pad(x, pads)


def kernel(feat, clusters, nbr_idx, proj_W, proj_g, proj_b, lw_W, lw_g, lw_b, wgt_W, adp_W, fuse_W, fuse_g, fuse_b, conv1_W, bn1_g, bn1_b, conv2_W, bn2_g, bn2_b):
    ids = jnp.pad(clusters.astype(jnp.int32), ((0, 0), (0, NPAD - N))).reshape(3, NGRP, 128)
    ids_off = ids + (jnp.arange(3, dtype=jnp.int32) * K)[:, None, None]
    vals3 = jnp.broadcast_to(
        jnp.pad(jnp.ones((N, 1), _f32), ((0, NPAD - N), (0, 0))), (NPAD, C)
    ) * jnp.ones((3, 1, 1), _f32)
    ztab = jnp.zeros((K, C), _f32)

    def seg_sum3(x3):
        x3p = _pad_rows(x3, NPAD)
        parts = _seg_scatter3(x3p, ids, ztab)
        return parts[0] + parts[1]

    def seg_broadcast3(tabs3):
        g = _seg_gather3(tabs3.reshape(3 * K, C), ids_off)
        return g[:, :N]

    Y1 = jax.nn.relu(_bn(jnp.einsum('nc,icd->ind', feat, lw_W), lw_g[:, None, :], lw_b[:, None, :], axis=1))
    cnt = _seg_counts3(ids, ztab, vals3)
    sums = seg_sum3(Y1)
    m = sums / jnp.clip(cnt, 1.0)[..., None]
    Gm = seg_broadcast3(m)
    pw = jnp.einsum('ind,ide->ine', Y1 - Gm, wgt_W)
    pw = jnp.exp(pw - jnp.max(pw, axis=(1, 2), keepdims=True))
    S = seg_sum3(pw)
    Sg = seg_broadcast3(S)
    pw = pw / (Sg + 1e-6)
    P = jax.nn.relu(_bn(jnp.einsum('nc,icd->ind', feat, proj_W[:3]), proj_g[:3, None, :], proj_b[:3, None, :], axis=1))
    pf = P * pw
    F = seg_sum3(pf)
    feats = seg_broadcast3(F)

    adp = jax.nn.softmax(feat @ adp_W, axis=1)
    agg = jnp.einsum('li,ilc->lc', adp, feats)
    f = jax.nn.relu(_bn(feat @ proj_W[3], proj_g[3], proj_b[3]))
    f = jnp.concatenate([f, agg], axis=1)
    f = jax.nn.relu(_bn(f @ fuse_W, fuse_g, fuse_b)) + feat
    res = f

    nbr_off = (jnp.pad(nbr_idx.astype(jnp.int32), ((0, NPAD - N), (0, 0))).T
               + (jnp.arange(NBR, dtype=jnp.int32) * NPAD)[:, None]).reshape(NBR, NGRP, 128)

    def subm_conv(x, W):
        y = jnp.einsum('nc,kcd->knd', _pad_rows(x, NPAD), W).reshape(NBR * NPAD, C)
        return _conv_gather_add(y, nbr_off)[:N]

    h = subm_conv(f, conv1_W)
    h = jax.nn.relu(_bn(h, bn1_g, bn1_b))
    h = subm_conv(h, conv2_W)
    h = _bn(h, bn2_g, bn2_b)
    return _final_relu_add(h, res)


# speedup vs baseline: 2.5014x; 1.2701x over previous
import functools

import jax
import jax.numpy as jnp
from jax import lax
from jax.experimental import pallas as pl
from jax.experimental.pallas import tpu as pltpu
from jax.experimental.pallas import tpu_sc as plsc

N = 50000
C = 128
K = 4096
NBR = 27
NPAD = 50176
NGRP = NPAD // 128
NW = 32
GPW = (NGRP + NW - 1) // NW
KR = K // 16

_MESH = plsc.VectorSubcoreMesh(core_axis_name="c", subcore_axis_name="s")
_f32 = jnp.float32


def _wid():
    return lax.axis_index("s") * 2 + lax.axis_index("c")


@functools.partial(
    pl.kernel,
    out_type=jax.ShapeDtypeStruct((2, 3, K, C), _f32),
    mesh=_MESH,
    scratch_types=(
        pltpu.VMEM((128,), jnp.int32),
        pltpu.VMEM((128, C), _f32),
        pltpu.VMEM_SHARED((K, C), _f32),
        pltpu.VMEM_SHARED((K, C), _f32),
        pltpu.VMEM_SHARED((K, C), _f32),
    ),
)
def _seg_scatter3(xs, ids, ztab, parts, idxb, rowb, t0, t1, t2):
    tabs = (t0, t1, t2)
    c = lax.axis_index("c")
    s = lax.axis_index("s")
    wid = _wid()
    for i in range(3):
        pltpu.sync_copy(ztab.at[pl.ds(s * KR, KR)], tabs[i].at[pl.ds(s * KR, KR)])
    plsc.subcore_barrier()

    def g_body(g, _):
        gid = wid + NW * g

        @pl.when(gid < NGRP)
        def _():
            base = gid * 128
            for i in range(3):
                pltpu.sync_copy(ids.at[i, gid], idxb)
                pltpu.sync_copy(xs.at[i, pl.ds(base, 128)], rowb)
                pltpu.sync_copy(rowb, tabs[i].at[idxb], add=True)
        return 0

    lax.fori_loop(0, GPW, g_body, 0)
    plsc.subcore_barrier()
    for i in range(3):
        pltpu.sync_copy(tabs[i].at[pl.ds(s * KR, KR)], parts.at[c, i, pl.ds(s * KR, KR)])


def _seg_counts3(ids, ztab, vals_ones3):
    parts = _seg_scatter3(vals_ones3, ids, ztab)
    return (parts[0] + parts[1])[..., 0]


@functools.partial(
    pl.kernel,
    out_type=jax.ShapeDtypeStruct((3, NPAD, C), _f32),
    mesh=_MESH,
    scratch_types=(
        pltpu.VMEM((128,), jnp.int32),
        pltpu.VMEM((128, C), _f32),
        pltpu.SemaphoreType.DMA,
    ),
)
def _seg_gather3(tabs, ids_off, out, idxb, rowb, sem):
    wid = _wid()

    def g_body(g, _):
        gid = wid + NW * g

        @pl.when(gid < NGRP)
        def _():
            base = gid * 128
            for i in range(3):
                pltpu.sync_copy(ids_off.at[i, gid], idxb)
                pltpu.async_copy(tabs.at[idxb], rowb, sem).wait()
                pltpu.sync_copy(rowb, out.at[i, pl.ds(base, 128)])
        return 0

    lax.fori_loop(0, GPW, g_body, 0)


@functools.partial(
    pl.kernel,
    out_type=jax.ShapeDtypeStruct((NPAD, C), _f32),
    mesh=_MESH,
    scratch_types=(
        pltpu.VMEM((128,), jnp.int32),
        pltpu.VMEM((128,), jnp.int32),
        pltpu.VMEM((128,), jnp.int32),
        pltpu.VMEM((128, C), _f32),
        pltpu.VMEM((128, C), _f32),
        pltpu.VMEM((128, C), _f32),
        pltpu.SemaphoreType.DMA,
        pltpu.SemaphoreType.DMA,
        pltpu.SemaphoreType.DMA,
    ),
)
def _conv_gather_add(yflat, nbr_off, out, idxa, idx0, idx1, accb, rowb0, rowb1,
                     sema, sem0, sem1):
    wid = _wid()

    def _acc(buf):
        def _acc_row(r, _):
            for cc in range(8):
                sl = pl.ds(cc * 16, 16)
                accb[r, sl] = accb[r, sl] + buf[r, sl]
            return 0

        lax.fori_loop(0, 128, _acc_row, 0)

    def g_body(g, _):
        gid = wid + NW * g

        @pl.when(gid < NGRP)
        def _():
            base = gid * 128
            pltpu.sync_copy(nbr_off.at[0, gid], idxa)
            pltpu.async_copy(yflat.at[idxa], accb, sema).wait()
            pltpu.sync_copy(nbr_off.at[1, gid], idx0)
            pltpu.async_copy(yflat.at[idx0], rowb0, sem0)

            def pair_body(j, _):
                kb = 2 * j + 2
                ka2 = 2 * j + 3

                @pl.when(kb < NBR)
                def _():
                    pltpu.sync_copy(nbr_off.at[kb, gid], idx1)
                    pltpu.async_copy(yflat.at[idx1], rowb1, sem1)

                pltpu.make_async_copy(yflat.at[idx0], rowb0, sem0).wait()
                _acc(rowb0)

                @pl.when(ka2 < NBR)
                def _():
                    pltpu.sync_copy(nbr_off.at[ka2, gid], idx0)
                    pltpu.async_copy(yflat.at[idx0], rowb0, sem0)

                @pl.when(kb < NBR)
                def _():
                    pltpu.make_async_copy(yflat.at[idx1], rowb1, sem1).wait()
                    _acc(rowb1)
                return 0

            lax.fori_loop(0, (NBR - 1 + 1) // 2, pair_body, 0)
            pltpu.sync_copy(accb, out.at[pl.ds(base, 128)])
        return 0

    lax.fori_loop(0, GPW, g_body, 0)


def _final_relu_add_kernel(h_ref, res_ref, out_ref):
    out_ref[...] = jnp.maximum(h_ref[...] + res_ref[...], 0.0)


def _final_relu_add(h, res):
    blk = 2000
    return pl.pallas_call(
        _final_relu_add_kernel,
        grid=(N // blk,),
        in_specs=[
            pl.BlockSpec((blk, C), lambda i: (i, 0)),
            pl.BlockSpec((blk, C), lambda i: (i, 0)),
        ],
        out_specs=pl.BlockSpec((blk, C), lambda i: (i, 0)),
        out_shape=jax.ShapeDtypeStruct((N, C), h.dtype),
    )(h, res)


def _bn(x, g, b, axis=0):
    m = jnp.mean(x, axis=axis, keepdims=True)
    v = jnp.var(x, axis=axis, keepdims=True)
    return (x - m) / jnp.sqrt(v + 1e-5) * g + b


def _pad_rows(x, npad):
    pads = [(0, 0)] * x.ndim
    pads[-2] = (0, npad - x.shape[-2])
    return jnp.pad(x, pads)


def kernel(feat, clusters, nbr_idx, proj_W, proj_g, proj_b, lw_W, lw_g, lw_b, wgt_W, adp_W, fuse_W, fuse_g, fuse_b, conv1_W, bn1_g, bn1_b, conv2_W, bn2_g, bn2_b):
    ids = jnp.pad(clusters.astype(jnp.int32), ((0, 0), (0, NPAD - N))).reshape(3, NGRP, 128)
    ids_off = ids + (jnp.arange(3, dtype=jnp.int32) * K)[:, None, None]
    vals3 = jnp.broadcast_to(
        jnp.pad(jnp.ones((N, 1), _f32), ((0, NPAD - N), (0, 0))), (NPAD, C)
    ) * jnp.ones((3, 1, 1), _f32)
    ztab = jnp.zeros((K, C), _f32)

    def seg_sum3(x3):
        x3p = _pad_rows(x3, NPAD)
        parts = _seg_scatter3(x3p, ids, ztab)
        return parts[0] + parts[1]

    def seg_broadcast3(tabs3):
        g = _seg_gather3(tabs3.reshape(3 * K, C), ids_off)
        return g[:, :N]

    Y1 = jax.nn.relu(_bn(jnp.einsum('nc,icd->ind', feat, lw_W), lw_g[:, None, :], lw_b[:, None, :], axis=1))
    cnt = _seg_counts3(ids, ztab, vals3)
    sums = seg_sum3(Y1)
    m = sums / jnp.clip(cnt, 1.0)[..., None]
    Gm = seg_broadcast3(m)
    pw = jnp.einsum('ind,ide->ine', Y1 - Gm, wgt_W)
    pw = jnp.exp(pw - jnp.max(pw, axis=(1, 2), keepdims=True))
    S = seg_sum3(pw)
    Sg = seg_broadcast3(S)
    pw = pw / (Sg + 1e-6)
    P = jax.nn.relu(_bn(jnp.einsum('nc,icd->ind', feat, proj_W[:3]), proj_g[:3, None, :], proj_b[:3, None, :], axis=1))
    pf = P * pw
    F = seg_sum3(pf)
    feats = seg_broadcast3(F)

    adp = jax.nn.softmax(feat @ adp_W, axis=1)
    agg = jnp.einsum('li,ilc->lc', adp, feats)
    f = jax.nn.relu(_bn(feat @ proj_W[3], proj_g[3], proj_b[3]))
    f = jnp.concatenate([f, agg], axis=1)
    f = jax.nn.relu(_bn(f @ fuse_W, fuse_g, fuse_b)) + feat
    res = f

    nbr_off = (jnp.pad(nbr_idx.astype(jnp.int32), ((0, NPAD - N), (0, 0))).T
               + (jnp.arange(NBR, dtype=jnp.int32) * NPAD)[:, None]).reshape(NBR, NGRP, 128)

    def subm_conv(x, W):
        y = jnp.einsum('nc,kcd->knd', _pad_rows(x, NPAD), W).reshape(NBR * NPAD, C)
        return _conv_gather_add(y, nbr_off)[:N]

    h = subm_conv(f, conv1_W)
    h = jax.nn.relu(_bn(h, bn1_g, bn1_b))
    h = subm_conv(h, conv2_W)
    h = _bn(h, bn2_g, bn2_b)
    return _final_relu_add(h, res)


# speedup vs baseline: 2.5939x; 1.0369x over previous
import functools

import jax
import jax.numpy as jnp
from jax import lax
from jax.experimental import pallas as pl
from jax.experimental.pallas import tpu as pltpu
from jax.experimental.pallas import tpu_sc as plsc

N = 50000
C = 128
K = 4096
NBR = 27
NPAD = 50176
NGRP = NPAD // 128
NW = 32
GPW = (NGRP + NW - 1) // NW
KR = K // 16
MG = N // 128
CUT = N - MG * 128
BLK = 512
GRID = NPAD // BLK
NEG = -3.4e38

_MESH = plsc.VectorSubcoreMesh(core_axis_name="c", subcore_axis_name="s")
_f32 = jnp.float32


def _wid():
    return lax.axis_index("s") * 2 + lax.axis_index("c")


@functools.partial(
    pl.kernel,
    out_type=jax.ShapeDtypeStruct((2, 3, K, C), _f32),
    mesh=_MESH,
    scratch_types=(
        pltpu.VMEM((128,), jnp.int32),
        pltpu.VMEM((128, C), _f32),
        pltpu.VMEM_SHARED((K, C), _f32),
        pltpu.VMEM_SHARED((K, C), _f32),
        pltpu.VMEM_SHARED((K, C), _f32),
    ),
)
def _seg_scatter3(xs, ids, ztab, parts, idxb, rowb, t0, t1, t2):
    tabs = (t0, t1, t2)
    c = lax.axis_index("c")
    s = lax.axis_index("s")
    wid = _wid()
    for i in range(3):
        pltpu.sync_copy(ztab.at[pl.ds(s * KR, KR)], tabs[i].at[pl.ds(s * KR, KR)])
    plsc.subcore_barrier()

    def g_body(g, _):
        gid = wid + NW * g

        @pl.when(gid < NGRP)
        def _():
            base = gid * 128
            for i in range(3):
                pltpu.sync_copy(ids.at[i, gid], idxb)
                pltpu.sync_copy(xs.at[i, pl.ds(base, 128)], rowb)
                pltpu.sync_copy(rowb, tabs[i].at[idxb], add=True)
        return 0

    lax.fori_loop(0, GPW, g_body, 0)
    plsc.subcore_barrier()
    for i in range(3):
        pltpu.sync_copy(tabs[i].at[pl.ds(s * KR, KR)], parts.at[c, i, pl.ds(s * KR, KR)])


@functools.partial(
    pl.kernel,
    out_type=jax.ShapeDtypeStruct((2, 3, K, C), _f32),
    mesh=_MESH,
    scratch_types=(
        pltpu.VMEM((128,), jnp.int32),
        pltpu.VMEM((128, C), _f32),
        pltpu.VMEM_SHARED((K, C), _f32),
        pltpu.VMEM_SHARED((K, C), _f32),
        pltpu.VMEM_SHARED((K, C), _f32),
    ),
)
def _cnt_scatter3(ids, ztab, cparts, idxb, valb, t0, t1, t2):
    tabs = (t0, t1, t2)
    c = lax.axis_index("c")
    s = lax.axis_index("s")
    wid = _wid()
    for i in range(3):
        pltpu.sync_copy(ztab.at[pl.ds(s * KR, KR)], tabs[i].at[pl.ds(s * KR, KR)])

    one = jnp.ones((16,), _f32)

    def fill_row(r, _):
        for cc in range(8):
            valb[r, pl.ds(cc * 16, 16)] = one
        return 0

    lax.fori_loop(0, 128, fill_row, 0)
    plsc.subcore_barrier()

    def g_body(g, _):
        gid = wid + NW * g

        @pl.when(gid == MG)
        def _():
            zero = jnp.zeros((16,), _f32)

            def zero_row(r, _):
                for cc in range(8):
                    valb[r, pl.ds(cc * 16, 16)] = zero
                return 0

            lax.fori_loop(CUT, 128, zero_row, 0)

        @pl.when(gid <= MG)
        def _():
            for i in range(3):
                pltpu.sync_copy(ids.at[i, gid], idxb)
                pltpu.sync_copy(valb, tabs[i].at[idxb], add=True)
        return 0

    lax.fori_loop(0, GPW, g_body, 0)
    plsc.subcore_barrier()
    for i in range(3):
        pltpu.sync_copy(tabs[i].at[pl.ds(s * KR, KR)], cparts.at[c, i, pl.ds(s * KR, KR)])


@functools.partial(
    pl.kernel,
    out_type=jax.ShapeDtypeStruct((3, NPAD, C), _f32),
    mesh=_MESH,
    scratch_types=(
        pltpu.VMEM((128,), jnp.int32),
        pltpu.VMEM((128, C), _f32),
        pltpu.SemaphoreType.DMA,
    ),
)
def _seg_gather3(tabs, ids_off, out, idxb, rowb, sem):
    wid = _wid()

    def g_body(g, _):
        gid = wid + NW * g

        @pl.when(gid < NGRP)
        def _():
            base = gid * 128
            for i in range(3):
                pltpu.sync_copy(ids_off.at[i, gid], idxb)
                pltpu.async_copy(tabs.at[idxb], rowb, sem).wait()
                pltpu.sync_copy(rowb, out.at[i, pl.ds(base, 128)])
        return 0

    lax.fori_loop(0, GPW, g_body, 0)


@functools.partial(
    pl.kernel,
    out_type=jax.ShapeDtypeStruct((NPAD, C), _f32),
    mesh=_MESH,
    scratch_types=(
        pltpu.VMEM((128,), jnp.int32),
        pltpu.VMEM((128,), jnp.int32),
        pltpu.VMEM((128,), jnp.int32),
        pltpu.VMEM((128, C), _f32),
        pltpu.VMEM((128, C), _f32),
        pltpu.VMEM((128, C), _f32),
        pltpu.SemaphoreType.DMA,
        pltpu.SemaphoreType.DMA,
        pltpu.SemaphoreType.DMA,
    ),
)
def _conv_gather_add(yflat, nbr_off, out, idxa, idx0, idx1, accb, rowb0, rowb1,
                     sema, sem0, sem1):
    wid = _wid()

    def _acc(buf):
        def _acc_row(r, _):
            for cc in range(8):
                sl = pl.ds(cc * 16, 16)
                accb[r, sl] = accb[r, sl] + buf[r, sl]
            return 0

        lax.fori_loop(0, 128, _acc_row, 0)

    def g_body(g, _):
        gid = wid + NW * g

        @pl.when(gid < NGRP)
        def _():
            base = gid * 128
            pltpu.sync_copy(nbr_off.at[0, gid], idxa)
            pltpu.async_copy(yflat.at[idxa], accb, sema).wait()
            pltpu.sync_copy(nbr_off.at[1, gid], idx0)
            pltpu.async_copy(yflat.at[idx0], rowb0, sem0)

            def pair_body(j, _):
                kb = 2 * j + 2
                ka2 = 2 * j + 3

                @pl.when(kb < NBR)
                def _():
                    pltpu.sync_copy(nbr_off.at[kb, gid], idx1)
                    pltpu.async_copy(yflat.at[idx1], rowb1, sem1)

                pltpu.make_async_copy(yflat.at[idx0], rowb0, sem0).wait()
                _acc(rowb0)

                @pl.when(ka2 < NBR)
                def _():
                    pltpu.sync_copy(nbr_off.at[ka2, gid], idx0)
                    pltpu.async_copy(yflat.at[idx0], rowb0, sem0)

                @pl.when(kb < NBR)
                def _():
                    pltpu.make_async_copy(yflat.at[idx1], rowb1, sem1).wait()
                    _acc(rowb1)
                return 0

            lax.fori_loop(0, NBR // 2, pair_body, 0)
            pltpu.sync_copy(accb, out.at[pl.ds(base, 128)])
        return 0

    lax.fori_loop(0, GPW, g_body, 0)


def _rmask(blk):
    return (lax.broadcasted_iota(jnp.int32, (blk, 1), 0)
            + pl.program_id(0) * blk) < N


def _dot(a, b):
    return jnp.dot(a, b, preferred_element_type=_f32)


def _stats1_kernel(feat_ref, ws_ref, out_ref):
    i = pl.program_id(0)
    x = jnp.where(_rmask(BLK), feat_ref[...], 0.0)
    rows = [jnp.sum(x, axis=0)[None]]
    for j in range(7):
        y = _dot(x, ws_ref[j])
        rows.append(jnp.sum(y * y, axis=0)[None])
    upd = jnp.concatenate(rows, axis=0)

    @pl.when(i == 0)
    def _():
        out_ref[...] = jnp.zeros_like(out_ref)

    out_ref[...] += upd


def _apply1_kernel(feat_ref, lw_ref, proj_ref, wgt_ref, sc_ref, sh_ref,
                   y1_ref, zp_ref, p_ref):
    x = feat_ref[...]
    mask = _rmask(BLK)
    for j in range(3):
        y = jnp.maximum(_dot(x, lw_ref[j]) * sc_ref[j] + sh_ref[j], 0.0)
        y = jnp.where(mask, y, 0.0)
        y1_ref[j] = y
        zp_ref[j] = _dot(y, wgt_ref[j])
    for j in range(4):
        p_ref[j] = jnp.maximum(_dot(x, proj_ref[j]) * sc_ref[3 + j] + sh_ref[3 + j], 0.0)


def _comb_mw_kernel(sp_ref, cp_ref, wgt_ref, out_ref):
    s = sp_ref[0, 0] + sp_ref[1, 0]
    c = cp_ref[0, 0] + cp_ref[1, 0]
    m = s / jnp.maximum(c, 1.0)
    out_ref[0] = _dot(m, wgt_ref[0])


def _comb_kernel(sp_ref, out_ref):
    out_ref[0] = sp_ref[0, 0] + sp_ref[1, 0]


def _zmax_kernel(zp_ref, gw_ref, z_ref, mx_ref):
    i = pl.program_id(0)
    z = zp_ref[...] - gw_ref[...]
    z_ref[...] = z
    mask = _rmask(BLK)[None]
    zm = jnp.where(mask, z, NEG)
    m3 = jnp.max(zm, axis=1)
    upd = jnp.concatenate([m3, jnp.full((5, C), NEG, _f32)], axis=0)

    @pl.when(i == 0)
    def _():
        mx_ref[...] = jnp.full_like(mx_ref, NEG)

    mx_ref[...] = jnp.maximum(mx_ref[...], upd)


def _exp_kernel(z_ref, m_ref, e_ref):
    m3 = m_ref[0:3][:, None, :]
    e = jnp.exp(z_ref[...] - m3)
    e_ref[...] = jnp.where(_rmask(BLK)[None], e, 0.0)


def _pf_kernel(p_ref, e_ref, sg_ref, out_ref):
    e = e_ref[...]
    pf = p_ref[...] * e / (sg_ref[...] + 1e-6)
    out_ref[...] = jnp.where(_rmask(BLK)[None], pf, 0.0)


def _fusion_kernel(feat_ref, p4_ref, fg_ref, adp_ref, wa_ref, wb_ref,
                   h_ref, st_ref):
    i = pl.program_id(0)
    x = feat_ref[...]
    logits = _dot(x, adp_ref[...])
    col = lax.broadcasted_iota(jnp.int32, (BLK, C), 1)
    logits = jnp.where(col < 3, logits, NEG)
    lmax = jnp.max(logits, axis=1, keepdims=True)
    ex = jnp.exp(logits - lmax)
    sm = ex / jnp.sum(ex, axis=1, keepdims=True)
    fg = fg_ref[...]
    agg = (sm[:, 0:1] * fg[0] + sm[:, 1:2] * fg[1] + sm[:, 2:3] * fg[2])
    h = _dot(p4_ref[0], wa_ref[...]) + _dot(agg, wb_ref[...])
    h_ref[...] = h
    mask = _rmask(BLK)
    hm = jnp.where(mask, h, 0.0)
    upd = jnp.concatenate(
        [jnp.sum(hm, axis=0)[None], jnp.sum(hm * hm, axis=0)[None],
         jnp.zeros((6, C), _f32)], axis=0)

    @pl.when(i == 0)
    def _():
        st_ref[...] = jnp.zeros_like(st_ref)

    st_ref[...] += upd


def _fused_conv_kernel(h_ref, feat_ref, ss_ref, w_ref, f_ref, y_ref):
    f = jnp.maximum(h_ref[...] * ss_ref[0] + ss_ref[1], 0.0) + feat_ref[...]
    f_ref[...] = f
    for k in range(NBR):
        y_ref[k] = _dot(f, w_ref[k])


def _colstats_kernel(x_ref, out_ref):
    i = pl.program_id(0)
    x = jnp.where(_rmask(BLK), x_ref[...], 0.0)
    upd = jnp.concatenate(
        [jnp.sum(x, axis=0)[None], jnp.sum(x * x, axis=0)[None],
         jnp.zeros((6, C), _f32)], axis=0)

    @pl.when(i == 0)
    def _():
        out_ref[...] = jnp.zeros_like(out_ref)

    out_ref[...] += upd


def _apply_conv_kernel(pre_ref, ss_ref, w_ref, y_ref):
    h = jnp.maximum(pre_ref[...] * ss_ref[0] + ss_ref[1], 0.0)
    for k in range(NBR):
        y_ref[k] = _dot(h, w_ref[k])


def _final_kernel(pre_ref, f_ref, ss_ref, out_ref):
    out_ref[...] = jnp.maximum(pre_ref[...] * ss_ref[0] + ss_ref[1] + f_ref[...], 0.0)


def _rowspec(nb=None, blk=BLK):
    if nb is None:
        return pl.BlockSpec((blk, C), lambda i: (i, 0))
    return pl.BlockSpec((nb, blk, C), lambda i: (0, i, 0))


def _fullspec(shape):
    nd = len(shape)
    return pl.BlockSpec(shape, lambda i, _n=nd: (0,) * _n)


def _sds(shape):
    return jax.ShapeDtypeStruct(shape, _f32)


def _scale_shift(g, b, mean, var):
    sc = g / jnp.sqrt(var + 1e-5)
    return sc, b - mean * sc


def _pack2(sc, sh):
    return jnp.concatenate([sc[None], sh[None], jnp.zeros((6, C), _f32)], axis=0)


def kernel(feat, clusters, nbr_idx, proj_W, proj_g, proj_b, lw_W, lw_g, lw_b, wgt_W, adp_W, fuse_W, fuse_g, fuse_b, conv1_W, bn1_g, bn1_b, conv2_W, bn2_g, bn2_b):
    ids = jnp.pad(clusters.astype(jnp.int32), ((0, 0), (0, NPAD - N))).reshape(3, NGRP, 128)
    ids_off = ids + (jnp.arange(3, dtype=jnp.int32) * K)[:, None, None]
    ztab = jnp.zeros((K, C), _f32)

    ws7 = jnp.concatenate([lw_W, proj_W], axis=0)
    st = pl.pallas_call(
        _stats1_kernel, grid=(GRID,),
        in_specs=[_rowspec(), _fullspec((7, C, C))],
        out_specs=_fullspec((8, C)),
        out_shape=_sds((8, C)),
    )(feat, ws7)
    mean_feat = st[0] / N
    means7 = mean_feat @ ws7
    var7 = st[1:8] / N - means7 * means7
    g7 = jnp.concatenate([lw_g, proj_g], axis=0)
    b7 = jnp.concatenate([lw_b, proj_b], axis=0)
    sc7, sh7 = _scale_shift(g7, b7, means7, var7)
    sc7 = jnp.concatenate([sc7, jnp.zeros((1, C), _f32)], axis=0)
    sh7 = jnp.concatenate([sh7, jnp.zeros((1, C), _f32)], axis=0)

    y1, zp, p = pl.pallas_call(
        _apply1_kernel, grid=(GRID,),
        in_specs=[_rowspec(), _fullspec((3, C, C)), _fullspec((4, C, C)),
                  _fullspec((3, C, C)), _fullspec((8, C)), _fullspec((8, C))],
        out_specs=(_rowspec(3), _rowspec(3), _rowspec(4)),
        out_shape=(_sds((3, NPAD, C)), _sds((3, NPAD, C)), _sds((4, NPAD, C))),
    )(feat, lw_W, proj_W, wgt_W, sc7, sh7)

    cparts = _cnt_scatter3(ids, ztab)
    sparts = _seg_scatter3(y1, ids, ztab)
    mw = pl.pallas_call(
        _comb_mw_kernel, grid=(3, K // BLK),
        in_specs=[pl.BlockSpec((2, 1, BLK, C), lambda b, k: (0, b, k, 0)),
                  pl.BlockSpec((2, 1, BLK, C), lambda b, k: (0, b, k, 0)),
                  pl.BlockSpec((1, C, C), lambda b, k: (b, 0, 0))],
        out_specs=pl.BlockSpec((1, BLK, C), lambda b, k: (b, k, 0)),
        out_shape=_sds((3, K, C)),
    )(sparts, cparts, wgt_W)
    gw = _seg_gather3(mw.reshape(3 * K, C), ids_off)

    z, mx = pl.pallas_call(
        _zmax_kernel, grid=(GRID,),
        in_specs=[_rowspec(3), _rowspec(3)],
        out_specs=(_rowspec(3), _fullspec((8, C))),
        out_shape=(_sds((3, NPAD, C)), _sds((8, C))),
    )(zp, gw)
    m3 = jnp.max(mx[0:3], axis=1)
    m8 = jnp.concatenate([jnp.broadcast_to(m3[:, None], (3, C)),
                          jnp.zeros((5, C), _f32)], axis=0)
    e = pl.pallas_call(
        _exp_kernel, grid=(GRID,),
        in_specs=[_rowspec(3), _fullspec((8, C))],
        out_specs=_rowspec(3),
        out_shape=_sds((3, NPAD, C)),
    )(z, m8)

    def comb(parts):
        return pl.pallas_call(
            _comb_kernel, grid=(3, K // BLK),
            in_specs=[pl.BlockSpec((2, 1, BLK, C), lambda b, k: (0, b, k, 0))],
            out_specs=pl.BlockSpec((1, BLK, C), lambda b, k: (b, k, 0)),
            out_shape=_sds((3, K, C)),
        )(parts)

    sg = _seg_gather3(comb(_seg_scatter3(e, ids, ztab)).reshape(3 * K, C), ids_off)
    pf = pl.pallas_call(
        _pf_kernel, grid=(GRID,),
        in_specs=[_rowspec(3), _rowspec(3), _rowspec(3)],
        out_specs=_rowspec(3),
        out_shape=_sds((3, NPAD, C)),
    )(p, e, sg)
    fg = _seg_gather3(comb(_seg_scatter3(pf, ids, ztab)).reshape(3 * K, C), ids_off)

    adp_wp = jnp.pad(adp_W, ((0, 0), (0, C - 3)))
    h, hst = pl.pallas_call(
        _fusion_kernel, grid=(GRID,),
        in_specs=[_rowspec(), pl.BlockSpec((1, BLK, C), lambda i: (3, i, 0)),
                  _rowspec(3), _fullspec((C, C)), _fullspec((C, C)),
                  _fullspec((C, C))],
        out_specs=(_rowspec(), _fullspec((8, C))),
        out_shape=(_sds((NPAD, C)), _sds((8, C))),
    )(feat, p, fg, adp_wp, fuse_W[:C], fuse_W[C:])
    fmean = hst[0] / N
    fvar = hst[1] / N - fmean * fmean
    fss = _pack2(*_scale_shift(fuse_g, fuse_b, fmean, fvar))

    f, yall1 = pl.pallas_call(
        _fused_conv_kernel, grid=(GRID,),
        in_specs=[_rowspec(), _rowspec(), _fullspec((8, C)),
                  _fullspec((NBR, C, C))],
        out_specs=(_rowspec(), _rowspec(NBR)),
        out_shape=(_sds((NPAD, C)), _sds((NBR, NPAD, C))),
    )(h, feat, fss, conv1_W)

    nbr_off = (jnp.pad(nbr_idx.astype(jnp.int32), ((0, NPAD - N), (0, 0))).T
               + (jnp.arange(NBR, dtype=jnp.int32) * NPAD)[:, None]).reshape(NBR, NGRP, 128)
    h1pre = _conv_gather_add(yall1.reshape(NBR * NPAD, C), nbr_off)

    def colstats(x):
        s = pl.pallas_call(
            _colstats_kernel, grid=(GRID,),
            in_specs=[_rowspec()],
            out_specs=_fullspec((8, C)),
            out_shape=_sds((8, C)),
        )(x)
        mean = s[0] / N
        return mean, s[1] / N - mean * mean

    b1ss = _pack2(*_scale_shift(bn1_g, bn1_b, *colstats(h1pre)))

    yall2 = pl.pallas_call(
        _apply_conv_kernel, grid=(GRID,),
        in_specs=[_rowspec(), _fullspec((8, C)), _fullspec((NBR, C, C))],
        out_specs=_rowspec(NBR),
        out_shape=_sds((NBR, NPAD, C)),
    )(h1pre, b1ss, conv2_W)
    h2pre = _conv_gather_add(yall2.reshape(NBR * NPAD, C), nbr_off)

    b2ss = _pack2(*_scale_shift(bn2_g, bn2_b, *colstats(h2pre)))
    return pl.pallas_call(
        _final_kernel, grid=(GRID,),
        in_specs=[_rowspec(), _rowspec(), _fullspec((8, C))],
        out_specs=_rowspec(),
        out_shape=_sds((N, C)),
    )(h2pre, f, b2ss)


# speedup vs baseline: 2.7804x; 1.0719x over previous
import functools

import jax
import jax.numpy as jnp
from jax import lax
from jax.experimental import pallas as pl
from jax.experimental.pallas import tpu as pltpu
from jax.experimental.pallas import tpu_sc as plsc

N = 50000
C = 128
K = 4096
NBR = 27
NPAD = 50176
NGRP = NPAD // 128
NW = 32
GPW = (NGRP + NW - 1) // NW
KR = K // 16
MG = N // 128
CUT = N - MG * 128
BLK = 512
GRID = NPAD // BLK
NEG = -3.4e38

_MESH = plsc.VectorSubcoreMesh(core_axis_name="c", subcore_axis_name="s")
_f32 = jnp.float32


def _wid():
    return lax.axis_index("s") * 2 + lax.axis_index("c")


@functools.partial(
    pl.kernel,
    out_type=jax.ShapeDtypeStruct((2, 3, K, C), _f32),
    mesh=_MESH,
    scratch_types=(
        pltpu.VMEM((128,), jnp.int32),
        pltpu.VMEM((128, C), _f32),
        pltpu.VMEM_SHARED((K, C), _f32),
        pltpu.VMEM_SHARED((K, C), _f32),
        pltpu.VMEM_SHARED((K, C), _f32),
        pltpu.SemaphoreType.DMA,
    ),
)
def _seg_scatter3(xs, ids, ztab, parts, idxb, rowb, t0, t1, t2, semr):
    tabs = (t0, t1, t2)
    c = lax.axis_index("c")
    s = lax.axis_index("s")
    wid = _wid()
    for i in range(3):
        pltpu.sync_copy(ztab.at[pl.ds(s * KR, KR)], tabs[i].at[pl.ds(s * KR, KR)])
    plsc.subcore_barrier()

    def g_body(g, _):
        gid = wid + NW * g

        @pl.when(gid < NGRP)
        def _():
            base = gid * 128
            dx = pltpu.async_copy(xs.at[0, pl.ds(base, 128)], rowb, semr)
            for i in range(3):
                pltpu.sync_copy(ids.at[i, gid], idxb)
                dx.wait()
                pltpu.sync_copy(rowb, tabs[i].at[idxb], add=True)
                if i < 2:
                    dx = pltpu.async_copy(xs.at[i + 1, pl.ds(base, 128)], rowb, semr)
        return 0

    lax.fori_loop(0, GPW, g_body, 0)
    plsc.subcore_barrier()
    for i in range(3):
        pltpu.sync_copy(tabs[i].at[pl.ds(s * KR, KR)], parts.at[c, i, pl.ds(s * KR, KR)])


@functools.partial(
    pl.kernel,
    out_type=jax.ShapeDtypeStruct((2, 3, K, C), _f32),
    mesh=_MESH,
    scratch_types=(
        pltpu.VMEM((128,), jnp.int32),
        pltpu.VMEM((128, C), _f32),
        pltpu.VMEM_SHARED((K, C), _f32),
        pltpu.VMEM_SHARED((K, C), _f32),
        pltpu.VMEM_SHARED((K, C), _f32),
    ),
)
def _cnt_scatter3(ids, ztab, cparts, idxb, valb, t0, t1, t2):
    tabs = (t0, t1, t2)
    c = lax.axis_index("c")
    s = lax.axis_index("s")
    wid = _wid()
    for i in range(3):
        pltpu.sync_copy(ztab.at[pl.ds(s * KR, KR)], tabs[i].at[pl.ds(s * KR, KR)])

    one = jnp.ones((16,), _f32)

    def fill_row(r, _):
        for cc in range(8):
            valb[r, pl.ds(cc * 16, 16)] = one
        return 0

    lax.fori_loop(0, 128, fill_row, 0)
    plsc.subcore_barrier()

    def g_body(g, _):
        gid = wid + NW * g

        @pl.when(gid == MG)
        def _():
            zero = jnp.zeros((16,), _f32)

            def zero_row(r, _):
                for cc in range(8):
                    valb[r, pl.ds(cc * 16, 16)] = zero
                return 0

            lax.fori_loop(CUT, 128, zero_row, 0)

        @pl.when(gid <= MG)
        def _():
            for i in range(3):
                pltpu.sync_copy(ids.at[i, gid], idxb)
                pltpu.sync_copy(valb, tabs[i].at[idxb], add=True)
        return 0

    lax.fori_loop(0, GPW, g_body, 0)
    plsc.subcore_barrier()
    for i in range(3):
        pltpu.sync_copy(tabs[i].at[pl.ds(s * KR, KR)], cparts.at[c, i, pl.ds(s * KR, KR)])


@functools.partial(
    pl.kernel,
    out_type=jax.ShapeDtypeStruct((3, NPAD, C), _f32),
    mesh=_MESH,
    scratch_types=(
        pltpu.VMEM((128,), jnp.int32),
        pltpu.VMEM((128,), jnp.int32),
        pltpu.VMEM((128,), jnp.int32),
        pltpu.VMEM((128, C), _f32),
        pltpu.VMEM((128, C), _f32),
        pltpu.VMEM((128, C), _f32),
        pltpu.SemaphoreType.DMA,
        pltpu.SemaphoreType.DMA,
        pltpu.SemaphoreType.DMA,
        pltpu.SemaphoreType.DMA,
    ),
)
def _seg_gather3(tabs, ids_off, out, i0, i1, i2, r0, r1, r2, s0, s1, s2, so):
    idxs = (i0, i1, i2)
    rows = (r0, r1, r2)
    sems = (s0, s1, s2)
    wid = _wid()

    def g_body(g, _):
        gid = wid + NW * g

        @pl.when(gid < NGRP)
        def _():
            base = gid * 128

            @pl.when(g > 0)
            def _():
                for i in range(3):
                    pltpu.make_async_copy(rows[i], out.at[i, pl.ds(0, 128)], so).wait()

            descs = []
            for i in range(3):
                pltpu.sync_copy(ids_off.at[i, gid], idxs[i])
                descs.append(pltpu.async_copy(tabs.at[idxs[i]], rows[i], sems[i]))
            for i in range(3):
                descs[i].wait()
                pltpu.async_copy(rows[i], out.at[i, pl.ds(base, 128)], so)
        return 0

    lax.fori_loop(0, GPW, g_body, 0)
    for i in range(3):
        pltpu.make_async_copy(rows[i], out.at[i, pl.ds(0, 128)], so).wait()


_bf16 = jnp.bfloat16


@functools.partial(
    pl.kernel,
    out_type=jax.ShapeDtypeStruct((NPAD, C), _f32),
    mesh=_MESH,
    scratch_types=(
        pltpu.VMEM((128,), jnp.int32),
        pltpu.VMEM((128,), jnp.int32),
        pltpu.VMEM((128,), jnp.int32),
        pltpu.VMEM((128, C), _f32),
        pltpu.VMEM((128, C), _f32),
        pltpu.VMEM((128, C), _f32),
        pltpu.SemaphoreType.DMA,
        pltpu.SemaphoreType.DMA,
        pltpu.SemaphoreType.DMA,
    ),
)
def _conv_gather_add(yflat, nbr_off, out, idxa, idx0, idx1, accb, rowb0, rowb1,
                     sema, sem0, sem1):
    wid = _wid()

    def _acc(buf):
        def _acc_row(r, _):
            for cc in range(8):
                sl = (r, pl.ds(cc * 16, 16))
                accb[sl] = accb[sl] + buf[sl]
            return 0

        lax.fori_loop(0, 128, _acc_row, 0)

    def g_body(g, _):
        gid = wid + NW * g

        @pl.when(gid < NGRP)
        def _():
            base = gid * 128
            pltpu.sync_copy(nbr_off.at[0, gid], idxa)
            pltpu.async_copy(yflat.at[idxa], accb, sema).wait()
            pltpu.sync_copy(nbr_off.at[1, gid], idx0)
            pltpu.async_copy(yflat.at[idx0], rowb0, sem0)

            def pair_body(j, _):
                kb = 2 * j + 2
                ka2 = 2 * j + 3

                @pl.when(kb < NBR)
                def _():
                    pltpu.sync_copy(nbr_off.at[kb, gid], idx1)
                    pltpu.async_copy(yflat.at[idx1], rowb1, sem1)

                pltpu.make_async_copy(yflat.at[idx0], rowb0, sem0).wait()
                _acc(rowb0)

                @pl.when(ka2 < NBR)
                def _():
                    pltpu.sync_copy(nbr_off.at[ka2, gid], idx0)
                    pltpu.async_copy(yflat.at[idx0], rowb0, sem0)

                @pl.when(kb < NBR)
                def _():
                    pltpu.make_async_copy(yflat.at[idx1], rowb1, sem1).wait()
                    _acc(rowb1)
                return 0

            lax.fori_loop(0, NBR // 2, pair_body, 0)
            pltpu.sync_copy(accb, out.at[pl.ds(base, 128)])
        return 0

    lax.fori_loop(0, GPW, g_body, 0)


def _rmask(blk):
    return (lax.broadcasted_iota(jnp.int32, (blk, 1), 0)
            + pl.program_id(0) * blk) < N


def _dot(a, b):
    return jnp.dot(a, b, preferred_element_type=_f32)


def _stats1_kernel(feat_ref, ws_ref, out_ref):
    i = pl.program_id(0)
    x = jnp.where(_rmask(BLK), feat_ref[...], 0.0)
    rows = [jnp.sum(x, axis=0)[None]]
    for j in range(7):
        y = _dot(x, ws_ref[j])
        rows.append(jnp.sum(y * y, axis=0)[None])
    upd = jnp.concatenate(rows, axis=0)

    @pl.when(i == 0)
    def _():
        out_ref[...] = jnp.zeros_like(out_ref)

    out_ref[...] += upd


def _apply1_kernel(feat_ref, lw_ref, proj_ref, wgt_ref, sc_ref, sh_ref,
                   y1_ref, zp_ref, p_ref):
    x = feat_ref[...]
    mask = _rmask(BLK)
    for j in range(3):
        y = jnp.maximum(_dot(x, lw_ref[j]) * sc_ref[j] + sh_ref[j], 0.0)
        y = jnp.where(mask, y, 0.0)
        y1_ref[j] = y
        zp_ref[j] = _dot(y, wgt_ref[j])
    for j in range(4):
        p_ref[j] = jnp.maximum(_dot(x, proj_ref[j]) * sc_ref[3 + j] + sh_ref[3 + j], 0.0)


def _comb_mw_kernel(sp_ref, cp_ref, wgt_ref, out_ref):
    s = sp_ref[0, 0] + sp_ref[1, 0]
    c = cp_ref[0, 0] + cp_ref[1, 0]
    m = s / jnp.maximum(c, 1.0)
    out_ref[0] = _dot(m, wgt_ref[0])


def _comb_kernel(sp_ref, out_ref):
    out_ref[0] = sp_ref[0, 0] + sp_ref[1, 0]


def _zmax_kernel(zp_ref, gw_ref, z_ref, mx_ref):
    i = pl.program_id(0)
    z = zp_ref[...] - gw_ref[...]
    z_ref[...] = z
    mask = _rmask(BLK)[None]
    zm = jnp.where(mask, z, NEG)
    m3 = jnp.max(zm, axis=1)
    upd = jnp.concatenate([m3, jnp.full((5, C), NEG, _f32)], axis=0)

    @pl.when(i == 0)
    def _():
        mx_ref[...] = jnp.full_like(mx_ref, NEG)

    mx_ref[...] = jnp.maximum(mx_ref[...], upd)


def _exp_kernel(z_ref, m_ref, e_ref):
    m3 = m_ref[0:3][:, None, :]
    e = jnp.exp(z_ref[...] - m3)
    e_ref[...] = jnp.where(_rmask(BLK)[None], e, 0.0)


def _pf_kernel(p_ref, e_ref, sg_ref, out_ref):
    e = e_ref[...]
    pf = p_ref[...] * e / (sg_ref[...] + 1e-6)
    out_ref[...] = jnp.where(_rmask(BLK)[None], pf, 0.0)


def _fusion_kernel(feat_ref, p4_ref, fg_ref, adp_ref, wa_ref, wb_ref,
                   h_ref, st_ref):
    i = pl.program_id(0)
    x = feat_ref[...]
    logits = _dot(x, adp_ref[...])
    col = lax.broadcasted_iota(jnp.int32, (BLK, C), 1)
    logits = jnp.where(col < 3, logits, NEG)
    lmax = jnp.max(logits, axis=1, keepdims=True)
    ex = jnp.exp(logits - lmax)
    sm = ex / jnp.sum(ex, axis=1, keepdims=True)
    fg = fg_ref[...]
    agg = (sm[:, 0:1] * fg[0] + sm[:, 1:2] * fg[1] + sm[:, 2:3] * fg[2])
    h = _dot(p4_ref[0], wa_ref[...]) + _dot(agg, wb_ref[...])
    h_ref[...] = h
    mask = _rmask(BLK)
    hm = jnp.where(mask, h, 0.0)
    upd = jnp.concatenate(
        [jnp.sum(hm, axis=0)[None], jnp.sum(hm * hm, axis=0)[None],
         jnp.zeros((6, C), _f32)], axis=0)

    @pl.when(i == 0)
    def _():
        st_ref[...] = jnp.zeros_like(st_ref)

    st_ref[...] += upd


def _fused_conv_kernel(h_ref, feat_ref, ss_ref, w_ref, f_ref, y_ref):
    f = jnp.maximum(h_ref[...] * ss_ref[0] + ss_ref[1], 0.0) + feat_ref[...]
    f_ref[...] = f
    for k in range(NBR):
        y_ref[k] = _dot(f, w_ref[k])


def _colstats_kernel(x_ref, out_ref):
    i = pl.program_id(0)
    x = jnp.where(_rmask(BLK), x_ref[...].astype(_f32), 0.0)
    upd = jnp.concatenate(
        [jnp.sum(x, axis=0)[None], jnp.sum(x * x, axis=0)[None],
         jnp.zeros((6, C), _f32)], axis=0)

    @pl.when(i == 0)
    def _():
        out_ref[...] = jnp.zeros_like(out_ref)

    out_ref[...] += upd


def _apply_conv_kernel(pre_ref, ss_ref, w_ref, y_ref):
    h = jnp.maximum(pre_ref[...].astype(_f32) * ss_ref[0] + ss_ref[1], 0.0)
    for k in range(NBR):
        y_ref[k] = _dot(h, w_ref[k])


def _final_kernel(pre_ref, f_ref, ss_ref, out_ref):
    out_ref[...] = jnp.maximum(
        pre_ref[...].astype(_f32) * ss_ref[0] + ss_ref[1] + f_ref[...], 0.0)


def _rowspec(nb=None, blk=BLK):
    if nb is None:
        return pl.BlockSpec((blk, C), lambda i: (i, 0))
    return pl.BlockSpec((nb, blk, C), lambda i: (0, i, 0))


def _fullspec(shape):
    nd = len(shape)
    return pl.BlockSpec(shape, lambda i, _n=nd: (0,) * _n)


def _sds(shape):
    return jax.ShapeDtypeStruct(shape, _f32)


def _scale_shift(g, b, mean, var):
    sc = g / jnp.sqrt(var + 1e-5)
    return sc, b - mean * sc


def _pack2(sc, sh):
    return jnp.concatenate([sc[None], sh[None], jnp.zeros((6, C), _f32)], axis=0)


def kernel(feat, clusters, nbr_idx, proj_W, proj_g, proj_b, lw_W, lw_g, lw_b, wgt_W, adp_W, fuse_W, fuse_g, fuse_b, conv1_W, bn1_g, bn1_b, conv2_W, bn2_g, bn2_b):
    ids = jnp.pad(clusters.astype(jnp.int32), ((0, 0), (0, NPAD - N))).reshape(3, NGRP, 128)
    ids_off = ids + (jnp.arange(3, dtype=jnp.int32) * K)[:, None, None]
    ztab = jnp.zeros((K, C), _f32)

    ws7 = jnp.concatenate([lw_W, proj_W], axis=0)
    st = pl.pallas_call(
        _stats1_kernel, grid=(GRID,),
        in_specs=[_rowspec(), _fullspec((7, C, C))],
        out_specs=_fullspec((8, C)),
        out_shape=_sds((8, C)),
    )(feat, ws7)
    mean_feat = st[0] / N
    means7 = mean_feat @ ws7
    var7 = st[1:8] / N - means7 * means7
    g7 = jnp.concatenate([lw_g, proj_g], axis=0)
    b7 = jnp.concatenate([lw_b, proj_b], axis=0)
    sc7, sh7 = _scale_shift(g7, b7, means7, var7)
    sc7 = jnp.concatenate([sc7, jnp.zeros((1, C), _f32)], axis=0)
    sh7 = jnp.concatenate([sh7, jnp.zeros((1, C), _f32)], axis=0)

    y1, zp, p = pl.pallas_call(
        _apply1_kernel, grid=(GRID,),
        in_specs=[_rowspec(), _fullspec((3, C, C)), _fullspec((4, C, C)),
                  _fullspec((3, C, C)), _fullspec((8, C)), _fullspec((8, C))],
        out_specs=(_rowspec(3), _rowspec(3), _rowspec(4)),
        out_shape=(_sds((3, NPAD, C)), _sds((3, NPAD, C)), _sds((4, NPAD, C))),
    )(feat, lw_W, proj_W, wgt_W, sc7, sh7)

    cparts = _cnt_scatter3(ids, ztab)
    sparts = _seg_scatter3(y1, ids, ztab)
    mw = pl.pallas_call(
        _comb_mw_kernel, grid=(3, K // BLK),
        in_specs=[pl.BlockSpec((2, 1, BLK, C), lambda b, k: (0, b, k, 0)),
                  pl.BlockSpec((2, 1, BLK, C), lambda b, k: (0, b, k, 0)),
                  pl.BlockSpec((1, C, C), lambda b, k: (b, 0, 0))],
        out_specs=pl.BlockSpec((1, BLK, C), lambda b, k: (b, k, 0)),
        out_shape=_sds((3, K, C)),
    )(sparts, cparts, wgt_W)
    gw = _seg_gather3(mw.reshape(3 * K, C), ids_off)

    z, mx = pl.pallas_call(
        _zmax_kernel, grid=(GRID,),
        in_specs=[_rowspec(3), _rowspec(3)],
        out_specs=(_rowspec(3), _fullspec((8, C))),
        out_shape=(_sds((3, NPAD, C)), _sds((8, C))),
    )(zp, gw)
    m3 = jnp.max(mx[0:3], axis=1)
    m8 = jnp.concatenate([jnp.broadcast_to(m3[:, None], (3, C)),
                          jnp.zeros((5, C), _f32)], axis=0)
    e = pl.pallas_call(
        _exp_kernel, grid=(GRID,),
        in_specs=[_rowspec(3), _fullspec((8, C))],
        out_specs=_rowspec(3),
        out_shape=_sds((3, NPAD, C)),
    )(z, m8)

    def comb(parts):
        return pl.pallas_call(
            _comb_kernel, grid=(3, K // BLK),
            in_specs=[pl.BlockSpec((2, 1, BLK, C), lambda b, k: (0, b, k, 0))],
            out_specs=pl.BlockSpec((1, BLK, C), lambda b, k: (b, k, 0)),
            out_shape=_sds((3, K, C)),
        )(parts)

    sg = _seg_gather3(comb(_seg_scatter3(e, ids, ztab)).reshape(3 * K, C), ids_off)
    pf = pl.pallas_call(
        _pf_kernel, grid=(GRID,),
        in_specs=[_rowspec(3), _rowspec(3), _rowspec(3)],
        out_specs=_rowspec(3),
        out_shape=_sds((3, NPAD, C)),
    )(p, e, sg)
    fg = _seg_gather3(comb(_seg_scatter3(pf, ids, ztab)).reshape(3 * K, C), ids_off)

    adp_wp = jnp.pad(adp_W, ((0, 0), (0, C - 3)))
    h, hst = pl.pallas_call(
        _fusion_kernel, grid=(GRID,),
        in_specs=[_rowspec(), pl.BlockSpec((1, BLK, C), lambda i: (3, i, 0)),
                  _rowspec(3), _fullspec((C, C)), _fullspec((C, C)),
                  _fullspec((C, C))],
        out_specs=(_rowspec(), _fullspec((8, C))),
        out_shape=(_sds((NPAD, C)), _sds((8, C))),
    )(feat, p, fg, adp_wp, fuse_W[:C], fuse_W[C:])
    fmean = hst[0] / N
    fvar = hst[1] / N - fmean * fmean
    fss = _pack2(*_scale_shift(fuse_g, fuse_b, fmean, fvar))

    f, yall1 = pl.pallas_call(
        _fused_conv_kernel, grid=(GRID,),
        in_specs=[_rowspec(), _rowspec(), _fullspec((8, C)),
                  _fullspec((NBR, C, C))],
        out_specs=(_rowspec(), _rowspec(NBR)),
        out_shape=(_sds((NPAD, C)), _sds((NBR, NPAD, C))),
    )(h, feat, fss, conv1_W)

    nbr_off = (jnp.pad(nbr_idx.astype(jnp.int32), ((0, NPAD - N), (0, 0))).T
               + (jnp.arange(NBR, dtype=jnp.int32) * NPAD)[:, None]).reshape(NBR, NGRP, 128)

    h1pre = _conv_gather_add(yall1.reshape(NBR * NPAD, C), nbr_off)

    def colstats(x):
        s = pl.pallas_call(
            _colstats_kernel, grid=(GRID,),
            in_specs=[_rowspec()],
            out_specs=_fullspec((8, C)),
            out_shape=_sds((8, C)),
        )(x)
        mean = s[0] / N
        return mean, s[1] / N - mean * mean

    b1ss = _pack2(*_scale_shift(bn1_g, bn1_b, *colstats(h1pre)))

    yall2 = pl.pallas_call(
        _apply_conv_kernel, grid=(GRID,),
        in_specs=[_rowspec(), _fullspec((8, C)), _fullspec((NBR, C, C))],
        out_specs=_rowspec(NBR),
        out_shape=_sds((NBR, NPAD, C)),
    )(h1pre, b1ss, conv2_W)
    h2pre = _conv_gather_add(yall2.reshape(NBR * NPAD, C), nbr_off)

    b2ss = _pack2(*_scale_shift(bn2_g, bn2_b, *colstats(h2pre)))
    return pl.pallas_call(
        _final_kernel, grid=(GRID,),
        in_specs=[_rowspec(), _rowspec(), _fullspec((8, C))],
        out_specs=_rowspec(),
        out_shape=_sds((N, C)),
    )(h2pre, f, b2ss)


# speedup vs baseline: 2.7836x; 1.0012x over previous
import functools

import jax
import jax.numpy as jnp
from jax import lax
from jax.experimental import pallas as pl
from jax.experimental.pallas import tpu as pltpu
from jax.experimental.pallas import tpu_sc as plsc

N = 50000
C = 128
K = 4096
NBR = 27
NPAD = 50176
NGRP = NPAD // 128
NW = 32
GPW = (NGRP + NW - 1) // NW
KR = K // 16
MG = N // 128
CUT = N - MG * 128
BLK = 512
GRID = NPAD // BLK
NEG = -3.4e38

_MESH = plsc.VectorSubcoreMesh(core_axis_name="c", subcore_axis_name="s")
_f32 = jnp.float32


def _wid():
    return lax.axis_index("s") * 2 + lax.axis_index("c")


@functools.partial(
    pl.kernel,
    out_type=jax.ShapeDtypeStruct((2, 3, K, C), _f32),
    mesh=_MESH,
    scratch_types=(
        pltpu.VMEM((128,), jnp.int32),
        pltpu.VMEM((128, C), _f32),
        pltpu.VMEM_SHARED((K, C), _f32),
        pltpu.VMEM_SHARED((K, C), _f32),
        pltpu.VMEM_SHARED((K, C), _f32),
        pltpu.SemaphoreType.DMA,
    ),
)
def _seg_scatter3(xs, ids, ztab, parts, idxb, rowb, t0, t1, t2, semr):
    tabs = (t0, t1, t2)
    c = lax.axis_index("c")
    s = lax.axis_index("s")
    wid = _wid()
    for i in range(3):
        pltpu.sync_copy(ztab.at[pl.ds(s * KR, KR)], tabs[i].at[pl.ds(s * KR, KR)])
    plsc.subcore_barrier()

    def g_body(g, _):
        gid = wid + NW * g

        @pl.when(gid < NGRP)
        def _():
            base = gid * 128
            dx = pltpu.async_copy(xs.at[0, pl.ds(base, 128)], rowb, semr)
            for i in range(3):
                pltpu.sync_copy(ids.at[i, gid], idxb)
                dx.wait()
                pltpu.sync_copy(rowb, tabs[i].at[idxb], add=True)
                if i < 2:
                    dx = pltpu.async_copy(xs.at[i + 1, pl.ds(base, 128)], rowb, semr)
        return 0

    lax.fori_loop(0, GPW, g_body, 0)
    plsc.subcore_barrier()
    for i in range(3):
        pltpu.sync_copy(tabs[i].at[pl.ds(s * KR, KR)], parts.at[c, i, pl.ds(s * KR, KR)])


@functools.partial(
    pl.kernel,
    out_type=jax.ShapeDtypeStruct((2, 3, K, C), _f32),
    mesh=_MESH,
    scratch_types=(
        pltpu.VMEM((128,), jnp.int32),
        pltpu.VMEM((128, C), _f32),
        pltpu.VMEM_SHARED((K, C), _f32),
        pltpu.VMEM_SHARED((K, C), _f32),
        pltpu.VMEM_SHARED((K, C), _f32),
    ),
)
def _cnt_scatter3(ids, ztab, cparts, idxb, valb, t0, t1, t2):
    tabs = (t0, t1, t2)
    c = lax.axis_index("c")
    s = lax.axis_index("s")
    wid = _wid()
    for i in range(3):
        pltpu.sync_copy(ztab.at[pl.ds(s * KR, KR)], tabs[i].at[pl.ds(s * KR, KR)])

    one = jnp.ones((16,), _f32)

    def fill_row(r, _):
        for cc in range(8):
            valb[r, pl.ds(cc * 16, 16)] = one
        return 0

    lax.fori_loop(0, 128, fill_row, 0)
    plsc.subcore_barrier()

    def g_body(g, _):
        gid = wid + NW * g

        @pl.when(gid == MG)
        def _():
            zero = jnp.zeros((16,), _f32)

            def zero_row(r, _):
                for cc in range(8):
                    valb[r, pl.ds(cc * 16, 16)] = zero
                return 0

            lax.fori_loop(CUT, 128, zero_row, 0)

        @pl.when(gid <= MG)
        def _():
            for i in range(3):
                pltpu.sync_copy(ids.at[i, gid], idxb)
                pltpu.sync_copy(valb, tabs[i].at[idxb], add=True)
        return 0

    lax.fori_loop(0, GPW, g_body, 0)
    plsc.subcore_barrier()
    for i in range(3):
        pltpu.sync_copy(tabs[i].at[pl.ds(s * KR, KR)], cparts.at[c, i, pl.ds(s * KR, KR)])


@functools.partial(
    pl.kernel,
    out_type=jax.ShapeDtypeStruct((3, NPAD, C), _f32),
    mesh=_MESH,
    scratch_types=(
        pltpu.VMEM((128,), jnp.int32),
        pltpu.VMEM((128,), jnp.int32),
        pltpu.VMEM((128,), jnp.int32),
        pltpu.VMEM((128, C), _f32),
        pltpu.VMEM((128, C), _f32),
        pltpu.VMEM((128, C), _f32),
        pltpu.SemaphoreType.DMA,
        pltpu.SemaphoreType.DMA,
        pltpu.SemaphoreType.DMA,
        pltpu.SemaphoreType.DMA,
    ),
)
def _seg_gather3(tabs, ids_off, out, i0, i1, i2, r0, r1, r2, s0, s1, s2, so):
    idxs = (i0, i1, i2)
    rows = (r0, r1, r2)
    sems = (s0, s1, s2)
    wid = _wid()

    def g_body(g, _):
        gid = wid + NW * g

        @pl.when(gid < NGRP)
        def _():
            base = gid * 128

            @pl.when(g > 0)
            def _():
                for i in range(3):
                    pltpu.make_async_copy(rows[i], out.at[i, pl.ds(0, 128)], so).wait()

            descs = []
            for i in range(3):
                pltpu.sync_copy(ids_off.at[i, gid], idxs[i])
                descs.append(pltpu.async_copy(tabs.at[idxs[i]], rows[i], sems[i]))
            for i in range(3):
                descs[i].wait()
                pltpu.async_copy(rows[i], out.at[i, pl.ds(base, 128)], so)
        return 0

    lax.fori_loop(0, GPW, g_body, 0)
    for i in range(3):
        pltpu.make_async_copy(rows[i], out.at[i, pl.ds(0, 128)], so).wait()


_bf16 = jnp.bfloat16


@functools.partial(
    pl.kernel,
    out_type=jax.ShapeDtypeStruct((NPAD, C), _f32),
    mesh=_MESH,
    scratch_types=(
        pltpu.VMEM((128,), jnp.int32),
        pltpu.VMEM((128,), jnp.int32),
        pltpu.VMEM((128,), jnp.int32),
        pltpu.VMEM((128, C), _f32),
        pltpu.VMEM((128, C), _f32),
        pltpu.VMEM((128, C), _f32),
        pltpu.SemaphoreType.DMA,
        pltpu.SemaphoreType.DMA,
        pltpu.SemaphoreType.DMA,
    ),
)
def _conv_gather_add(yflat, nbr_off, out, idxa, idx0, idx1, accb, rowb0, rowb1,
                     sema, sem0, sem1):
    wid = _wid()

    def _acc(buf):
        def _acc_row(r, _):
            for rr in range(4):
                for cc in range(8):
                    sl = (r * 4 + rr, pl.ds(cc * 16, 16))
                    accb[sl] = accb[sl] + buf[sl]
            return 0

        lax.fori_loop(0, 32, _acc_row, 0)

    def g_body(g, _):
        gid = wid + NW * g

        @pl.when(gid < NGRP)
        def _():
            base = gid * 128
            pltpu.sync_copy(nbr_off.at[0, gid], idxa)
            pltpu.async_copy(yflat.at[idxa], accb, sema).wait()
            pltpu.sync_copy(nbr_off.at[1, gid], idx0)
            pltpu.async_copy(yflat.at[idx0], rowb0, sem0)

            def pair_body(j, _):
                kb = 2 * j + 2
                ka2 = 2 * j + 3

                @pl.when(kb < NBR)
                def _():
                    pltpu.sync_copy(nbr_off.at[kb, gid], idx1)
                    pltpu.async_copy(yflat.at[idx1], rowb1, sem1)

                pltpu.make_async_copy(yflat.at[idx0], rowb0, sem0).wait()
                _acc(rowb0)

                @pl.when(ka2 < NBR)
                def _():
                    pltpu.sync_copy(nbr_off.at[ka2, gid], idx0)
                    pltpu.async_copy(yflat.at[idx0], rowb0, sem0)

                @pl.when(kb < NBR)
                def _():
                    pltpu.make_async_copy(yflat.at[idx1], rowb1, sem1).wait()
                    _acc(rowb1)
                return 0

            lax.fori_loop(0, NBR // 2, pair_body, 0)
            pltpu.sync_copy(accb, out.at[pl.ds(base, 128)])
        return 0

    lax.fori_loop(0, GPW, g_body, 0)


def _rmask(blk):
    return (lax.broadcasted_iota(jnp.int32, (blk, 1), 0)
            + pl.program_id(0) * blk) < N


def _dot(a, b):
    return jnp.dot(a, b, preferred_element_type=_f32)


def _stats1_kernel(feat_ref, ws_ref, out_ref):
    i = pl.program_id(0)
    x = jnp.where(_rmask(BLK), feat_ref[...], 0.0)
    rows = [jnp.sum(x, axis=0)[None]]
    for j in range(7):
        y = _dot(x, ws_ref[j])
        rows.append(jnp.sum(y * y, axis=0)[None])
    upd = jnp.concatenate(rows, axis=0)

    @pl.when(i == 0)
    def _():
        out_ref[...] = jnp.zeros_like(out_ref)

    out_ref[...] += upd


def _apply1_kernel(feat_ref, lw_ref, proj_ref, wgt_ref, sc_ref, sh_ref,
                   y1_ref, zp_ref, p_ref):
    x = feat_ref[...]
    mask = _rmask(BLK)
    for j in range(3):
        y = jnp.maximum(_dot(x, lw_ref[j]) * sc_ref[j] + sh_ref[j], 0.0)
        y = jnp.where(mask, y, 0.0)
        y1_ref[j] = y
        zp_ref[j] = _dot(y, wgt_ref[j])
    for j in range(4):
        p_ref[j] = jnp.maximum(_dot(x, proj_ref[j]) * sc_ref[3 + j] + sh_ref[3 + j], 0.0)


def _comb_mw_kernel(sp_ref, cp_ref, wgt_ref, out_ref):
    s = sp_ref[0, 0] + sp_ref[1, 0]
    c = cp_ref[0, 0] + cp_ref[1, 0]
    m = s / jnp.maximum(c, 1.0)
    out_ref[0] = _dot(m, wgt_ref[0])


def _comb_kernel(sp_ref, out_ref):
    out_ref[0] = sp_ref[0, 0] + sp_ref[1, 0]


def _zmax_kernel(zp_ref, gw_ref, z_ref, mx_ref):
    i = pl.program_id(0)
    z = zp_ref[...] - gw_ref[...]
    z_ref[...] = z
    mask = _rmask(BLK)[None]
    zm = jnp.where(mask, z, NEG)
    m3 = jnp.max(zm, axis=1)
    upd = jnp.concatenate([m3, jnp.full((5, C), NEG, _f32)], axis=0)

    @pl.when(i == 0)
    def _():
        mx_ref[...] = jnp.full_like(mx_ref, NEG)

    mx_ref[...] = jnp.maximum(mx_ref[...], upd)


def _exp_kernel(z_ref, m_ref, e_ref):
    m3 = m_ref[0:3][:, None, :]
    e = jnp.exp(z_ref[...] - m3)
    e_ref[...] = jnp.where(_rmask(BLK)[None], e, 0.0)


def _pf_kernel(p_ref, e_ref, sg_ref, out_ref):
    e = e_ref[...]
    pf = p_ref[...] * e / (sg_ref[...] + 1e-6)
    out_ref[...] = jnp.where(_rmask(BLK)[None], pf, 0.0)


def _fusion_kernel(feat_ref, p4_ref, fg_ref, adp_ref, wa_ref, wb_ref,
                   h_ref, st_ref):
    i = pl.program_id(0)
    x = feat_ref[...]
    logits = _dot(x, adp_ref[...])
    col = lax.broadcasted_iota(jnp.int32, (BLK, C), 1)
    logits = jnp.where(col < 3, logits, NEG)
    lmax = jnp.max(logits, axis=1, keepdims=True)
    ex = jnp.exp(logits - lmax)
    sm = ex / jnp.sum(ex, axis=1, keepdims=True)
    fg = fg_ref[...]
    agg = (sm[:, 0:1] * fg[0] + sm[:, 1:2] * fg[1] + sm[:, 2:3] * fg[2])
    h = _dot(p4_ref[0], wa_ref[...]) + _dot(agg, wb_ref[...])
    h_ref[...] = h
    mask = _rmask(BLK)
    hm = jnp.where(mask, h, 0.0)
    upd = jnp.concatenate(
        [jnp.sum(hm, axis=0)[None], jnp.sum(hm * hm, axis=0)[None],
         jnp.zeros((6, C), _f32)], axis=0)

    @pl.when(i == 0)
    def _():
        st_ref[...] = jnp.zeros_like(st_ref)

    st_ref[...] += upd


def _fused_conv_kernel(h_ref, feat_ref, ss_ref, w_ref, f_ref, y_ref):
    f = jnp.maximum(h_ref[...] * ss_ref[0] + ss_ref[1], 0.0) + feat_ref[...]
    f_ref[...] = f
    for k in range(NBR):
        y_ref[k] = _dot(f, w_ref[k])


def _colstats_kernel(x_ref, out_ref):
    i = pl.program_id(0)
    x = jnp.where(_rmask(BLK), x_ref[...].astype(_f32), 0.0)
    upd = jnp.concatenate(
        [jnp.sum(x, axis=0)[None], jnp.sum(x * x, axis=0)[None],
         jnp.zeros((6, C), _f32)], axis=0)

    @pl.when(i == 0)
    def _():
        out_ref[...] = jnp.zeros_like(out_ref)

    out_ref[...] += upd


def _apply_conv_kernel(pre_ref, ss_ref, w_ref, y_ref):
    h = jnp.maximum(pre_ref[...].astype(_f32) * ss_ref[0] + ss_ref[1], 0.0)
    for k in range(NBR):
        y_ref[k] = _dot(h, w_ref[k])


def _final_kernel(pre_ref, f_ref, ss_ref, out_ref):
    out_ref[...] = jnp.maximum(
        pre_ref[...].astype(_f32) * ss_ref[0] + ss_ref[1] + f_ref[...], 0.0)


def _rowspec(nb=None, blk=BLK):
    if nb is None:
        return pl.BlockSpec((blk, C), lambda i: (i, 0))
    return pl.BlockSpec((nb, blk, C), lambda i: (0, i, 0))


def _fullspec(shape):
    nd = len(shape)
    return pl.BlockSpec(shape, lambda i, _n=nd: (0,) * _n)


def _sds(shape):
    return jax.ShapeDtypeStruct(shape, _f32)


def _scale_shift(g, b, mean, var):
    sc = g / jnp.sqrt(var + 1e-5)
    return sc, b - mean * sc


def _pack2(sc, sh):
    return jnp.concatenate([sc[None], sh[None], jnp.zeros((6, C), _f32)], axis=0)


def kernel(feat, clusters, nbr_idx, proj_W, proj_g, proj_b, lw_W, lw_g, lw_b, wgt_W, adp_W, fuse_W, fuse_g, fuse_b, conv1_W, bn1_g, bn1_b, conv2_W, bn2_g, bn2_b):
    ids = jnp.pad(clusters.astype(jnp.int32), ((0, 0), (0, NPAD - N))).reshape(3, NGRP, 128)
    ids_off = ids + (jnp.arange(3, dtype=jnp.int32) * K)[:, None, None]
    ztab = jnp.zeros((K, C), _f32)

    ws7 = jnp.concatenate([lw_W, proj_W], axis=0)
    st = pl.pallas_call(
        _stats1_kernel, grid=(GRID,),
        in_specs=[_rowspec(), _fullspec((7, C, C))],
        out_specs=_fullspec((8, C)),
        out_shape=_sds((8, C)),
    )(feat, ws7)
    mean_feat = st[0] / N
    means7 = mean_feat @ ws7
    var7 = st[1:8] / N - means7 * means7
    g7 = jnp.concatenate([lw_g, proj_g], axis=0)
    b7 = jnp.concatenate([lw_b, proj_b], axis=0)
    sc7, sh7 = _scale_shift(g7, b7, means7, var7)
    sc7 = jnp.concatenate([sc7, jnp.zeros((1, C), _f32)], axis=0)
    sh7 = jnp.concatenate([sh7, jnp.zeros((1, C), _f32)], axis=0)

    y1, zp, p = pl.pallas_call(
        _apply1_kernel, grid=(GRID,),
        in_specs=[_rowspec(), _fullspec((3, C, C)), _fullspec((4, C, C)),
                  _fullspec((3, C, C)), _fullspec((8, C)), _fullspec((8, C))],
        out_specs=(_rowspec(3), _rowspec(3), _rowspec(4)),
        out_shape=(_sds((3, NPAD, C)), _sds((3, NPAD, C)), _sds((4, NPAD, C))),
    )(feat, lw_W, proj_W, wgt_W, sc7, sh7)

    cparts = _cnt_scatter3(ids, ztab)
    sparts = _seg_scatter3(y1, ids, ztab)
    mw = pl.pallas_call(
        _comb_mw_kernel, grid=(3, K // BLK),
        in_specs=[pl.BlockSpec((2, 1, BLK, C), lambda b, k: (0, b, k, 0)),
                  pl.BlockSpec((2, 1, BLK, C), lambda b, k: (0, b, k, 0)),
                  pl.BlockSpec((1, C, C), lambda b, k: (b, 0, 0))],
        out_specs=pl.BlockSpec((1, BLK, C), lambda b, k: (b, k, 0)),
        out_shape=_sds((3, K, C)),
    )(sparts, cparts, wgt_W)
    gw = _seg_gather3(mw.reshape(3 * K, C), ids_off)

    z, mx = pl.pallas_call(
        _zmax_kernel, grid=(GRID,),
        in_specs=[_rowspec(3), _rowspec(3)],
        out_specs=(_rowspec(3), _fullspec((8, C))),
        out_shape=(_sds((3, NPAD, C)), _sds((8, C))),
    )(zp, gw)
    m3 = jnp.max(mx[0:3], axis=1)
    m8 = jnp.concatenate([jnp.broadcast_to(m3[:, None], (3, C)),
                          jnp.zeros((5, C), _f32)], axis=0)
    e = pl.pallas_call(
        _exp_kernel, grid=(GRID,),
        in_specs=[_rowspec(3), _fullspec((8, C))],
        out_specs=_rowspec(3),
        out_shape=_sds((3, NPAD, C)),
    )(z, m8)

    def comb(parts):
        return pl.pallas_call(
            _comb_kernel, grid=(3, K // BLK),
            in_specs=[pl.BlockSpec((2, 1, BLK, C), lambda b, k: (0, b, k, 0))],
            out_specs=pl.BlockSpec((1, BLK, C), lambda b, k: (b, k, 0)),
            out_shape=_sds((3, K, C)),
        )(parts)

    sg = _seg_gather3(comb(_seg_scatter3(e, ids, ztab)).reshape(3 * K, C), ids_off)
    pf = pl.pallas_call(
        _pf_kernel, grid=(GRID,),
        in_specs=[_rowspec(3), _rowspec(3), _rowspec(3)],
        out_specs=_rowspec(3),
        out_shape=_sds((3, NPAD, C)),
    )(p, e, sg)
    fg = _seg_gather3(comb(_seg_scatter3(pf, ids, ztab)).reshape(3 * K, C), ids_off)

    adp_wp = jnp.pad(adp_W, ((0, 0), (0, C - 3)))
    h, hst = pl.pallas_call(
        _fusion_kernel, grid=(GRID,),
        in_specs=[_rowspec(), pl.BlockSpec((1, BLK, C), lambda i: (3, i, 0)),
                  _rowspec(3), _fullspec((C, C)), _fullspec((C, C)),
                  _fullspec((C, C))],
        out_specs=(_rowspec(), _fullspec((8, C))),
        out_shape=(_sds((NPAD, C)), _sds((8, C))),
    )(feat, p, fg, adp_wp, fuse_W[:C], fuse_W[C:])
    fmean = hst[0] / N
    fvar = hst[1] / N - fmean * fmean
    fss = _pack2(*_scale_shift(fuse_g, fuse_b, fmean, fvar))

    f, yall1 = pl.pallas_call(
        _fused_conv_kernel, grid=(GRID,),
        in_specs=[_rowspec(), _rowspec(), _fullspec((8, C)),
                  _fullspec((NBR, C, C))],
        out_specs=(_rowspec(), _rowspec(NBR)),
        out_shape=(_sds((NPAD, C)), _sds((NBR, NPAD, C))),
    )(h, feat, fss, conv1_W)

    nbr_off = (jnp.pad(nbr_idx.astype(jnp.int32), ((0, NPAD - N), (0, 0))).T
               + (jnp.arange(NBR, dtype=jnp.int32) * NPAD)[:, None]).reshape(NBR, NGRP, 128)

    h1pre = _conv_gather_add(yall1.reshape(NBR * NPAD, C), nbr_off)

    def colstats(x):
        s = pl.pallas_call(
            _colstats_kernel, grid=(GRID,),
            in_specs=[_rowspec()],
            out_specs=_fullspec((8, C)),
            out_shape=_sds((8, C)),
        )(x)
        mean = s[0] / N
        return mean, s[1] / N - mean * mean

    b1ss = _pack2(*_scale_shift(bn1_g, bn1_b, *colstats(h1pre)))

    yall2 = pl.pallas_call(
        _apply_conv_kernel, grid=(GRID,),
        in_specs=[_rowspec(), _fullspec((8, C)), _fullspec((NBR, C, C))],
        out_specs=_rowspec(NBR),
        out_shape=_sds((NBR, NPAD, C)),
    )(h1pre, b1ss, conv2_W)
    h2pre = _conv_gather_add(yall2.reshape(NBR * NPAD, C), nbr_off)

    b2ss = _pack2(*_scale_shift(bn2_g, bn2_b, *colstats(h2pre)))
    return pl.pallas_call(
        _final_kernel, grid=(GRID,),
        in_specs=[_rowspec(), _rowspec(), _fullspec((8, C))],
        out_specs=_rowspec(),
        out_shape=_sds((N, C)),
    )(h2pre, f, b2ss)
